# baseline (speedup 1.0000x reference)
import jax
import jax.numpy as jnp
from jax.experimental import pallas as pl

_S = 8192
_LOG2E = 1.4426950408889634
_N = 128


def _body(x_ref, ew_ref, eb_ref, w1_ref, b1_ref, w2_ref, b2_ref,
          w3_ref, b3_ref, mu_ref, o_ref):
    f32 = jnp.float32
    w1 = w1_ref[...]
    v1 = jnp.dot(w1, ew_ref[...], preferred_element_type=f32)
    c1 = jnp.dot(w1, eb_ref[...], preferred_element_type=f32) + b1_ref[...]
    w3m = w3_ref[...] * _LOG2E
    b3m = b3_ref[...] * _LOG2E
    b3c = b3m - jnp.max(b3m)
    u = jnp.max(w3m, axis=0, keepdims=True)
    p2 = jnp.concatenate([mu_ref[...].reshape(1, _N),
                          jnp.ones((1, _N), f32)], axis=0)

    bf16 = jnp.bfloat16
    w2h = w2_ref[...].astype(bf16)
    w3h = w3m.astype(bf16)
    xr = x_ref[...].reshape(1, _S)
    h = jnp.dot(v1, xr, preferred_element_type=f32) + c1
    h = jnp.maximum(h, 0.0).astype(bf16)
    h = jnp.dot(w2h, h, preferred_element_type=f32) + b2_ref[...]
    h = jnp.maximum(h, 0.0).astype(bf16)
    l = jnp.dot(w3h, h, preferred_element_type=f32)
    m = jnp.dot(u.astype(bf16), h, preferred_element_type=f32)
    e = jnp.exp2(jnp.maximum(l + b3c - m, -100.0))
    r = jnp.dot(p2, e, preferred_element_type=f32)
    o_ref[...] = (r[0:1, :] / r[1:2, :]).reshape(1, 1, _S)


def kernel(x, enc_w, enc_b, W1, b1, W2, b2, W3, b3, mu_proj):
    B = x.shape[0]
    N = enc_w.shape[1]
    grid = (B // _S,)
    x3 = x.reshape(B // _S, 1, _S)
    ewc = enc_w.reshape(N, 1)
    ebc = enc_b.reshape(N, 1)
    b1c = b1.reshape(N, 1)
    b2c = b2.reshape(N, 1)
    b3c = b3.reshape(N, 1)

    full = lambda shp: pl.BlockSpec(shp, lambda i: tuple(0 for _ in shp))
    out = pl.pallas_call(
        _body,
        grid=grid,
        in_specs=[
            pl.BlockSpec((1, 1, _S), lambda i: (i, 0, 0)),
            full(ewc.shape),
            full(ebc.shape),
            full(W1.shape), full(b1c.shape),
            full(W2.shape), full(b2c.shape),
            full(W3.shape), full(b3c.shape),
            full(mu_proj.shape),
        ],
        out_specs=pl.BlockSpec((1, 1, _S), lambda i: (i, 0, 0)),
        out_shape=jax.ShapeDtypeStruct((B // _S, 1, _S), jnp.float32),
    )(x3, ewc, ebc, W1, b1c, W2, b2c, W3, b3c, mu_proj)
    return out.reshape(B, 1)


# speedup vs baseline: 1.1287x; 1.1287x over previous
import jax
import jax.numpy as jnp
from jax.experimental import pallas as pl

_S = 8192
_LOG2E = 1.4426950408889634
_N = 128


def _body(x_ref, ew_ref, eb_ref, w1_ref, b1_ref, w2_ref, b2_ref,
          w3_ref, b3_ref, mu_ref, o_ref):
    f32 = jnp.float32
    w1 = w1_ref[...]
    v1 = jnp.dot(w1, ew_ref[...], preferred_element_type=f32)
    c1 = jnp.dot(w1, eb_ref[...], preferred_element_type=f32) + b1_ref[...]
    w3m = w3_ref[...] * _LOG2E
    b3m = b3_ref[...] * _LOG2E
    b3c = b3m - jnp.max(b3m)
    u = jnp.max(w3m, axis=0, keepdims=True)
    w3d = (w3m - u).astype(jnp.bfloat16)
    p2 = jnp.concatenate([mu_ref[...].reshape(1, _N),
                          jnp.ones((1, _N), f32)], axis=0)
    vc = jnp.concatenate([v1, c1], axis=1)
    w2h = w2_ref[...].astype(jnp.bfloat16)

    bf16 = jnp.bfloat16
    xr = x_ref[...].reshape(1, _S)
    x2 = jnp.concatenate([xr, jnp.ones((1, _S), f32)], axis=0)
    h = jnp.dot(vc, x2, preferred_element_type=f32)
    h = jnp.maximum(h, 0.0).astype(bf16)
    h = jnp.dot(w2h, h, preferred_element_type=f32) + b2_ref[...]
    h = jnp.maximum(h, 0.0).astype(bf16)
    l = jnp.dot(w3d, h, preferred_element_type=f32) + b3c
    e = jnp.exp2(jnp.maximum(l, -100.0))
    r = jnp.dot(p2, e, preferred_element_type=f32)
    o_ref[...] = (r[0:1, :] / r[1:2, :]).reshape(1, 1, _S)


def kernel(x, enc_w, enc_b, W1, b1, W2, b2, W3, b3, mu_proj):
    B = x.shape[0]
    N = enc_w.shape[1]
    grid = (B // _S,)
    x3 = x.reshape(B // _S, 1, _S)
    ewc = enc_w.reshape(N, 1)
    ebc = enc_b.reshape(N, 1)
    b1c = b1.reshape(N, 1)
    b2c = b2.reshape(N, 1)
    b3c = b3.reshape(N, 1)

    full = lambda shp: pl.BlockSpec(shp, lambda i: tuple(0 for _ in shp))
    out = pl.pallas_call(
        _body,
        grid=grid,
        in_specs=[
            pl.BlockSpec((1, 1, _S), lambda i: (i, 0, 0)),
            full(ewc.shape),
            full(ebc.shape),
            full(W1.shape), full(b1c.shape),
            full(W2.shape), full(b2c.shape),
            full(W3.shape), full(b3c.shape),
            full(mu_proj.shape),
        ],
        out_specs=pl.BlockSpec((1, 1, _S), lambda i: (i, 0, 0)),
        out_shape=jax.ShapeDtypeStruct((B // _S, 1, _S), jnp.float32),
    )(x3, ewc, ebc, W1, b1c, W2, b2c, W3, b3c, mu_proj)
    return out.reshape(B, 1)


# speedup vs baseline: 1.2302x; 1.0899x over previous
import jax
import jax.numpy as jnp
from jax.experimental import pallas as pl

_S = 16384
_LOG2E = 1.4426950408889634
_N = 128


def _body(x_ref, ew_ref, eb_ref, w1_ref, b1_ref, w2_ref, b2_ref,
          w3_ref, b3_ref, mu_ref, o_ref):
    f32 = jnp.float32
    w1 = w1_ref[...]
    v1 = jnp.dot(w1, ew_ref[...], preferred_element_type=f32)
    c1 = jnp.dot(w1, eb_ref[...], preferred_element_type=f32) + b1_ref[...]
    w3m = w3_ref[...] * _LOG2E
    b3m = b3_ref[...] * _LOG2E
    b3c = b3m - jnp.max(b3m)
    u = jnp.max(w3m, axis=0, keepdims=True)
    w3d = (w3m - u).astype(jnp.bfloat16)
    s3 = jnp.exp2(b3c).reshape(1, _N)
    p2 = jnp.concatenate([mu_ref[...].reshape(1, _N) * s3,
                          s3], axis=0)
    vc = jnp.concatenate([v1, c1], axis=1)
    w2h = w2_ref[...].astype(jnp.bfloat16)

    bf16 = jnp.bfloat16
    xr = x_ref[...].reshape(1, _S)
    x2 = jnp.concatenate([xr, jnp.ones((1, _S), f32)], axis=0)
    h = jnp.dot(vc, x2, preferred_element_type=f32)
    h = jnp.maximum(h, 0.0).astype(bf16)
    h = jnp.dot(w2h, h, preferred_element_type=f32) + b2_ref[...]
    h = jnp.maximum(h, 0.0).astype(bf16)
    l = jnp.dot(w3d, h, preferred_element_type=f32)
    e = jnp.exp2(jnp.maximum(l, -100.0))
    r = jnp.dot(p2, e, preferred_element_type=f32)
    o_ref[...] = (r[0:1, :] / r[1:2, :]).reshape(1, 1, _S)


def kernel(x, enc_w, enc_b, W1, b1, W2, b2, W3, b3, mu_proj):
    B = x.shape[0]
    N = enc_w.shape[1]
    grid = (B // _S,)
    x3 = x.reshape(B // _S, 1, _S)
    ewc = enc_w.reshape(N, 1)
    ebc = enc_b.reshape(N, 1)
    b1c = b1.reshape(N, 1)
    b2c = b2.reshape(N, 1)
    b3c = b3.reshape(N, 1)

    full = lambda shp: pl.BlockSpec(shp, lambda i: tuple(0 for _ in shp))
    out = pl.pallas_call(
        _body,
        grid=grid,
        in_specs=[
            pl.BlockSpec((1, 1, _S), lambda i: (i, 0, 0)),
            full(ewc.shape),
            full(ebc.shape),
            full(W1.shape), full(b1c.shape),
            full(W2.shape), full(b2c.shape),
            full(W3.shape), full(b3c.shape),
            full(mu_proj.shape),
        ],
        out_specs=pl.BlockSpec((1, 1, _S), lambda i: (i, 0, 0)),
        out_shape=jax.ShapeDtypeStruct((B // _S, 1, _S), jnp.float32),
    )(x3, ewc, ebc, W1, b1c, W2, b2c, W3, b3c, mu_proj)
    return out.reshape(B, 1)
